# Initial kernel scaffold; baseline (speedup 1.0000x reference)
import jax
import jax.numpy as jnp
from jax.experimental import pallas as pl


def kernel(classifications, regressions, anchors, annotations, image):
    raise NotImplementedError("write your pallas kernel here")


# speedup vs baseline: 1.4293x; 1.4293x over previous
import functools

import jax
import jax.numpy as jnp
from jax.experimental import pallas as pl

_ALPHA = 0.25
_GAMMA_IS_SQUARE = True
_A_TOTAL = 49104
_BA = 4096


def _focal_body(ann_ref, cls_ref, reg_ref, anc_ref, out_ref):
    j = pl.program_id(0)
    i = pl.program_id(1)

    @pl.when(jnp.logical_and(j == 0, i == 0))
    def _init():
        out_ref[...] = jnp.zeros_like(out_ref)

    bx1 = ann_ref[0, 0:1, :]
    by1 = ann_ref[0, 1:2, :]
    bx2 = ann_ref[0, 2:3, :]
    by2 = ann_ref[0, 3:4, :]
    bcl = ann_ref[0, 4:5, :]

    ax1 = anc_ref[0, :, 0:1]
    ay1 = anc_ref[0, :, 1:2]
    ax2 = anc_ref[0, :, 2:3]
    ay2 = anc_ref[0, :, 3:4]

    iw = jnp.clip(jnp.minimum(ax2, bx2) - jnp.maximum(ax1, bx1), 0.0, None)
    ih = jnp.clip(jnp.minimum(ay2, by2) - jnp.maximum(ay1, by1), 0.0, None)
    inter = iw * ih
    area_b = (bx2 - bx1) * (by2 - by1)
    area_a = (ax2 - ax1) * (ay2 - ay1)
    ua = jnp.clip(area_a + area_b - inter, 1e-8, None)
    iou = inter / ua
    iou = jnp.where(bcl != -1.0, iou, -1.0)

    iou_max = jnp.max(iou, axis=1, keepdims=True)
    g_iota = jax.lax.broadcasted_iota(jnp.int32, iou.shape, 1)
    first_arg = jnp.min(jnp.where(iou >= iou_max, g_iota, 2**30), axis=1,
                        keepdims=True)
    eq = (g_iota == first_arg).astype(jnp.float32)
    gx1 = jnp.sum(eq * bx1, axis=1, keepdims=True)
    gy1 = jnp.sum(eq * by1, axis=1, keepdims=True)
    gx2 = jnp.sum(eq * bx2, axis=1, keepdims=True)
    gy2 = jnp.sum(eq * by2, axis=1, keepdims=True)
    gcl = jnp.sum(eq * bcl, axis=1, keepdims=True)

    positive = iou_max >= 0.5
    low = iou_max < 0.4
    row_idx = jax.lax.broadcasted_iota(jnp.int32, iou_max.shape, 0) + i * _BA
    rv = row_idx < _A_TOTAL
    posv = jnp.logical_and(positive, rv)

    aw = ax2 - ax1
    ah = ay2 - ay1
    acx = ax1 + 0.5 * aw
    acy = ay1 + 0.5 * ah
    gwr = gx2 - gx1
    ghr = gy2 - gy1
    gcx = gx1 + 0.5 * gwr
    gcy = gy1 + 0.5 * ghr
    gw = jnp.clip(gwr, 1.0, None)
    gh = jnp.clip(ghr, 1.0, None)
    tdx = (gcx - acx) / aw * 10.0
    tdy = (gcy - acy) / ah * 10.0
    tdw = jnp.log(gw / aw) * 5.0
    tdh = jnp.log(gh / ah) * 5.0

    def _sl1(d):
        ad = jnp.abs(d)
        return jnp.where(ad <= 1.0 / 9.0, 4.5 * ad * ad, ad - 0.5 / 9.0)

    reg = reg_ref[0]
    rl = (_sl1(tdx - reg[:, 0:1]) + _sl1(tdy - reg[:, 1:2])
          + _sl1(tdw - reg[:, 2:3]) + _sl1(tdh - reg[:, 3:4]))
    reg_part = jnp.sum(jnp.where(posv, rl, 0.0))

    c = jnp.clip(cls_ref[0], 1e-4, 1.0 - 1e-4)
    neg = (1.0 - _ALPHA) * c * c * (-jnp.log(1.0 - c))
    neg_sum = jnp.sum(neg, axis=1, keepdims=True)
    c_iota = jax.lax.broadcasted_iota(jnp.int32, c.shape, 1)
    ct = jnp.sum(jnp.where(c_iota == gcl.astype(jnp.int32), c, 0.0),
                 axis=1, keepdims=True)
    ct = jnp.clip(ct, 1e-4, 1.0 - 1e-4)
    pos_t = _ALPHA * (1.0 - ct) * (1.0 - ct) * (-jnp.log(ct))
    neg_t = (1.0 - _ALPHA) * ct * ct * (-jnp.log(1.0 - ct))
    row_cls = (jnp.where(low, neg_sum, 0.0)
               + jnp.where(positive, neg_sum - neg_t + pos_t, 0.0))
    cls_part = jnp.sum(jnp.where(rv, row_cls, 0.0))
    pos_part = jnp.sum(jnp.where(posv, 1.0, 0.0))

    r8 = jax.lax.broadcasted_iota(jnp.int32, (8, 128), 0)
    c128 = jax.lax.broadcasted_iota(jnp.int32, (8, 128), 1)
    sel = r8 == j
    upd = (jnp.where(jnp.logical_and(sel, c128 == 0), cls_part, 0.0)
           + jnp.where(jnp.logical_and(sel, c128 == 1), reg_part, 0.0)
           + jnp.where(jnp.logical_and(sel, c128 == 2), pos_part, 0.0))
    out_ref[...] += upd


@functools.partial(jax.jit, static_argnames=("interpret",))
def kernel(classifications, regressions, anchors, annotations, image,
           interpret=False):
    del image
    B, A, C = classifications.shape
    G = annotations.shape[1]
    nb = (A + _BA - 1) // _BA
    ann_t = jnp.swapaxes(annotations, 1, 2)
    acc = pl.pallas_call(
        _focal_body,
        grid=(B, nb),
        in_specs=[
            pl.BlockSpec((1, 5, G), lambda j, i: (j, 0, 0)),
            pl.BlockSpec((1, _BA, C), lambda j, i: (j, i, 0)),
            pl.BlockSpec((1, _BA, 4), lambda j, i: (j, i, 0)),
            pl.BlockSpec((1, _BA, 4), lambda j, i: (0, i, 0)),
        ],
        out_specs=pl.BlockSpec((8, 128), lambda j, i: (0, 0)),
        out_shape=jax.ShapeDtypeStruct((8, 128), jnp.float32),
        interpret=interpret,
    )(ann_t, classifications, regressions, anchors)
    cls_sum = acc[0:B, 0]
    reg_sum = acc[0:B, 1]
    npos = acc[0:B, 2]
    cls_loss = cls_sum / jnp.maximum(npos, 1.0)
    reg_loss = reg_sum / jnp.maximum(npos * 4.0, 1.0)
    return (jnp.mean(cls_loss, keepdims=True),
            jnp.mean(reg_loss, keepdims=True))


# speedup vs baseline: 2.2144x; 1.5493x over previous
import functools

import jax
import jax.numpy as jnp
from jax.experimental import pallas as pl
from jax.experimental.pallas import tpu as pltpu

_ALPHA = 0.25
_A_TOTAL = 49104
_BA = 4096
_G = 20


def _assign_body(ann_ref, anc_ref, reg_ref, keep_ref, pos_ref, cls_ref,
                 acc_ref):
    j = pl.program_id(0)
    sa = anc_ref.shape[1]

    ax1 = anc_ref[0]
    ay1 = anc_ref[1]
    ax2 = anc_ref[2]
    ay2 = anc_ref[3]
    area_a = (ax2 - ax1) * (ay2 - ay1)

    best = jnp.full((sa, 128), -2.0, jnp.float32)
    gx1 = jnp.zeros((sa, 128), jnp.float32)
    gy1 = gx1
    gx2 = gx1
    gy2 = gx1
    gcl = gx1
    for g in range(_G):
        bx1 = ann_ref[j, g, 0]
        by1 = ann_ref[j, g, 1]
        bx2 = ann_ref[j, g, 2]
        by2 = ann_ref[j, g, 3]
        bcl = ann_ref[j, g, 4]
        iw = jnp.clip(jnp.minimum(ax2, bx2) - jnp.maximum(ax1, bx1), 0.0, None)
        ih = jnp.clip(jnp.minimum(ay2, by2) - jnp.maximum(ay1, by1), 0.0, None)
        inter = iw * ih
        ua = jnp.clip(area_a + (bx2 - bx1) * (by2 - by1) - inter, 1e-8, None)
        iou = jnp.where(bcl != -1.0, inter / ua, -1.0)
        upd = iou > best
        best = jnp.where(upd, iou, best)
        gx1 = jnp.where(upd, bx1, gx1)
        gy1 = jnp.where(upd, by1, gy1)
        gx2 = jnp.where(upd, bx2, gx2)
        gy2 = jnp.where(upd, by2, gy2)
        gcl = jnp.where(upd, bcl, gcl)

    lane = jax.lax.broadcasted_iota(jnp.int32, (sa, 128), 1)
    sub = jax.lax.broadcasted_iota(jnp.int32, (sa, 128), 0)
    rv = sub * 128 + lane < _A_TOTAL
    positive = best >= 0.5
    posv = jnp.logical_and(positive, rv)
    keep = jnp.logical_and(jnp.logical_or(best < 0.4, positive), rv)

    keep_ref[0] = jnp.where(keep, 1.0, 0.0)
    pos_ref[0] = jnp.where(posv, 1.0, 0.0)
    cls_ref[0] = gcl

    aw = ax2 - ax1
    ah = ay2 - ay1
    gwr = gx2 - gx1
    ghr = gy2 - gy1
    tdx = (gx1 + 0.5 * gwr - ax1 - 0.5 * aw) / aw * 10.0
    tdy = (gy1 + 0.5 * ghr - ay1 - 0.5 * ah) / ah * 10.0
    tdw = jnp.log(jnp.clip(gwr, 1.0, None) / aw) * 5.0
    tdh = jnp.log(jnp.clip(ghr, 1.0, None) / ah) * 5.0

    def _sl1(d):
        ad = jnp.abs(d)
        return jnp.where(ad <= 1.0 / 9.0, 4.5 * ad * ad, ad - 0.5 / 9.0)

    rl = (_sl1(tdx - reg_ref[0, 0]) + _sl1(tdy - reg_ref[0, 1])
          + _sl1(tdw - reg_ref[0, 2]) + _sl1(tdh - reg_ref[0, 3]))
    reg_part = jnp.sum(jnp.where(posv, rl, 0.0))
    pos_part = jnp.sum(jnp.where(posv, 1.0, 0.0))
    c128 = jax.lax.broadcasted_iota(jnp.int32, (8, 128), 1)
    acc_ref[0] = (jnp.where(c128 == 1, reg_part, 0.0)
                  + jnp.where(c128 == 2, pos_part, 0.0))


def _cls_body(cls_ref, keep_ref, pos_ref, tcls_ref, out_ref):
    i = pl.program_id(1)

    @pl.when(i == 0)
    def _init():
        out_ref[...] = jnp.zeros_like(out_ref)

    kc = keep_ref[0] > 0.5
    pc = pos_ref[0] > 0.5
    ic = tcls_ref[0].astype(jnp.int32)

    c = jnp.clip(cls_ref[0], 1e-4, 1.0 - 1e-4)
    neg = (1.0 - _ALPHA) * c * c * (-jnp.log(1.0 - c))
    c_iota = jax.lax.broadcasted_iota(jnp.int32, neg.shape, 1)
    onehot = c_iota == ic
    s1 = jnp.sum(jnp.where(kc, neg, 0.0))
    s_negt = jnp.sum(jnp.where(jnp.logical_and(onehot, pc), neg, 0.0))
    ct = jnp.sum(jnp.where(onehot, c, 0.0), axis=1, keepdims=True)
    ct = jnp.clip(ct, 1e-4, 1.0 - 1e-4)
    post = _ALPHA * (1.0 - ct) * (1.0 - ct) * (-jnp.log(ct))
    s_post = jnp.sum(jnp.where(pc, post, 0.0))
    cls_part = s1 - s_negt + s_post

    c128 = jax.lax.broadcasted_iota(jnp.int32, (8, 128), 1)
    out_ref[0] += jnp.where(c128 == 0, cls_part, 0.0)


@functools.partial(jax.jit, static_argnames=("interpret",))
def kernel(classifications, regressions, anchors, annotations, image,
           interpret=False):
    del image
    B, A, C = classifications.shape
    nb = (A + _BA - 1) // _BA
    a_pad = nb * _BA
    sa = a_pad // 128
    anc_t = jnp.swapaxes(anchors[0], 0, 1)
    anc_t = jnp.reshape(jnp.pad(anc_t, ((0, 0), (0, a_pad - A))),
                        (4, sa, 128))
    reg_t = jnp.swapaxes(regressions, 1, 2)
    reg_t = jnp.reshape(jnp.pad(reg_t, ((0, 0), (0, 0), (0, a_pad - A))),
                        (B, 4, sa, 128))

    keep_f, pos_f, cls_f, acc_a = pl.pallas_call(
        _assign_body,
        grid=(B,),
        in_specs=[
            pl.BlockSpec(memory_space=pltpu.SMEM),
            pl.BlockSpec((4, sa, 128), lambda j: (0, 0, 0)),
            pl.BlockSpec((1, 4, sa, 128), lambda j: (j, 0, 0, 0)),
        ],
        out_specs=[
            pl.BlockSpec((1, sa, 128), lambda j: (j, 0, 0)),
            pl.BlockSpec((1, sa, 128), lambda j: (j, 0, 0)),
            pl.BlockSpec((1, sa, 128), lambda j: (j, 0, 0)),
            pl.BlockSpec((1, 8, 128), lambda j: (j, 0, 0)),
        ],
        out_shape=[
            jax.ShapeDtypeStruct((B, sa, 128), jnp.float32),
            jax.ShapeDtypeStruct((B, sa, 128), jnp.float32),
            jax.ShapeDtypeStruct((B, sa, 128), jnp.float32),
            jax.ShapeDtypeStruct((B, 8, 128), jnp.float32),
        ],
        compiler_params=pltpu.CompilerParams(
            dimension_semantics=("parallel",)),
        interpret=interpret,
    )(annotations, anc_t, reg_t)

    keep_c = jnp.reshape(keep_f, (B, a_pad, 1))
    pos_c = jnp.reshape(pos_f, (B, a_pad, 1))
    cls_c = jnp.reshape(cls_f, (B, a_pad, 1))

    acc_b = pl.pallas_call(
        _cls_body,
        grid=(B, nb),
        in_specs=[
            pl.BlockSpec((1, _BA, C), lambda j, i: (j, i, 0)),
            pl.BlockSpec((1, _BA, 1), lambda j, i: (j, i, 0)),
            pl.BlockSpec((1, _BA, 1), lambda j, i: (j, i, 0)),
            pl.BlockSpec((1, _BA, 1), lambda j, i: (j, i, 0)),
        ],
        out_specs=pl.BlockSpec((1, 8, 128), lambda j, i: (j, 0, 0)),
        out_shape=jax.ShapeDtypeStruct((B, 8, 128), jnp.float32),
        compiler_params=pltpu.CompilerParams(
            dimension_semantics=("parallel", "arbitrary")),
        interpret=interpret,
    )(classifications, keep_c, pos_c, cls_c)

    cls_sum = acc_b[:, 0, 0]
    reg_sum = acc_a[:, 0, 1]
    npos = acc_a[:, 0, 2]
    cls_loss = cls_sum / jnp.maximum(npos, 1.0)
    reg_loss = reg_sum / jnp.maximum(npos * 4.0, 1.0)
    return (jnp.mean(cls_loss, keepdims=True),
            jnp.mean(reg_loss, keepdims=True))
